# baseline (speedup 1.0000x reference)
import functools

import jax
import jax.numpy as jnp
from jax import lax
from jax.experimental import pallas as pl
from jax.experimental.pallas import tpu as pltpu

N_ATOMS = 10000
N_EDGES = 320000
D = 128
N_RBF = 20
N_INT = 3
MAX_ZN = 100
CUTOFF = 5.0
_LOG2 = 0.6931471805599453

BE = 1600
NEB = N_EDGES // BE
BA = 1000
NAB = N_ATOMS // BA


def _ssp(x):
    return jnp.maximum(x, 0.0) + jnp.log(1.0 + jnp.exp(-jnp.abs(x))) - _LOG2


def _edge_filter_body(r_blk, fw1, fb1, fw2, fb2, out_blk):
    r = r_blk[...]
    d = jnp.sqrt(jnp.sum(r * r, axis=1, keepdims=True))
    width = CUTOFF / (N_RBF - 1)
    offs = lax.broadcasted_iota(jnp.int32, (1, N_RBF), 1).astype(jnp.float32) * width
    coeff = -0.5 / (width * width)
    fij = jnp.exp(coeff * (d - offs) ** 2)
    rcut = 0.5 * (jnp.cos(d * (jnp.pi / CUTOFF)) + 1.0)
    rcut = rcut * (d < CUTOFF).astype(jnp.float32)
    t = _ssp(jnp.dot(fij, fw1[...], preferred_element_type=jnp.float32) + fb1[...])
    w = jnp.dot(t, fw2[...], preferred_element_type=jnp.float32) + fb2[...]
    out_blk[...] = w * rcut


def _edge_filter(r_ij, fw1l, fb1l, fw2l, fb2l):
    return pl.pallas_call(
        _edge_filter_body,
        grid=(NEB,),
        in_specs=[
            pl.BlockSpec((BE, 3), lambda i: (i, 0)),
            pl.BlockSpec((N_RBF, D), lambda i: (0, 0)),
            pl.BlockSpec((1, D), lambda i: (0, 0)),
            pl.BlockSpec((D, D), lambda i: (0, 0)),
            pl.BlockSpec((1, D), lambda i: (0, 0)),
        ],
        out_specs=pl.BlockSpec((BE, D), lambda i: (i, 0)),
        out_shape=jax.ShapeDtypeStruct((N_EDGES, D), jnp.float32),
    )(r_ij, fw1l, fb1l, fw2l, fb2l)


def _embed_body(az_blk, emb, in2f0, x0_blk, h0_blk):
    az = az_blk[...]
    ids = lax.broadcasted_iota(jnp.int32, (BA, MAX_ZN), 1)
    onehot = (az == ids).astype(jnp.float32)
    x0 = jnp.dot(onehot, emb[...], preferred_element_type=jnp.float32)
    x0_blk[...] = x0
    h0_blk[...] = jnp.dot(x0, in2f0[...], preferred_element_type=jnp.float32)


def _embed(az2, emb, in2f0):
    return pl.pallas_call(
        _embed_body,
        grid=(NAB,),
        in_specs=[
            pl.BlockSpec((BA, 1), lambda i: (i, 0)),
            pl.BlockSpec((MAX_ZN, D), lambda i: (0, 0)),
            pl.BlockSpec((D, D), lambda i: (0, 0)),
        ],
        out_specs=[
            pl.BlockSpec((BA, D), lambda i: (i, 0)),
            pl.BlockSpec((BA, D), lambda i: (i, 0)),
        ],
        out_shape=[
            jax.ShapeDtypeStruct((N_ATOMS, D), jnp.float32),
            jax.ShapeDtypeStruct((N_ATOMS, D), jnp.float32),
        ],
    )(az2, emb, in2f0)


def _node_body(parts, x_blk, oW1, ob1, oW2, ob2, in2f_next, xn_blk, hn_blk):
    agg = parts[0, :, :] + parts[1, :, :]
    t = _ssp(jnp.dot(agg, oW1[...], preferred_element_type=jnp.float32) + ob1[...])
    v = jnp.dot(t, oW2[...], preferred_element_type=jnp.float32) + ob2[...]
    xn = x_blk[...] + v
    xn_blk[...] = xn
    hn_blk[...] = jnp.dot(xn, in2f_next[...], preferred_element_type=jnp.float32)


def _node(parts, x, oW1, ob1, oW2, ob2, in2f_next):
    return pl.pallas_call(
        _node_body,
        grid=(NAB,),
        in_specs=[
            pl.BlockSpec((2, BA, D), lambda i: (0, i, 0)),
            pl.BlockSpec((BA, D), lambda i: (i, 0)),
            pl.BlockSpec((D, D), lambda i: (0, 0)),
            pl.BlockSpec((1, D), lambda i: (0, 0)),
            pl.BlockSpec((D, D), lambda i: (0, 0)),
            pl.BlockSpec((1, D), lambda i: (0, 0)),
            pl.BlockSpec((D, D), lambda i: (0, 0)),
        ],
        out_specs=[
            pl.BlockSpec((BA, D), lambda i: (i, 0)),
            pl.BlockSpec((BA, D), lambda i: (i, 0)),
        ],
        out_shape=[
            jax.ShapeDtypeStruct((N_ATOMS, D), jnp.float32),
            jax.ShapeDtypeStruct((N_ATOMS, D), jnp.float32),
        ],
    )(parts, x, oW1, ob1, oW2, ob2, in2f_next)


def kernel(atomic_numbers, r_ij, idx_i, idx_j, idx, idx_m, emb, in2f_W,
           fw1, fb1, fw2, fb2, o_W1, o_b1, o_W2, o_b2):
    az2 = atomic_numbers.reshape(N_ATOMS, 1).astype(jnp.int32)
    x, h = _embed(az2, emb, in2f_W[0])
    for l in range(N_INT):
        wij = _edge_filter(r_ij, fw1[l], fb1[l].reshape(1, D),
                           fw2[l], fb2[l].reshape(1, D))
        xij = h[idx_j] * wij
        agg = jax.ops.segment_sum(xij, idx_i, num_segments=N_ATOMS)
        parts = jnp.stack([agg, jnp.zeros_like(agg)])
        x, h = _node(parts, x, o_W1[l], o_b1[l].reshape(1, D),
                     o_W2[l], o_b2[l].reshape(1, D), in2f_W[(l + 1) % N_INT])
    return x


# speedup vs baseline: 1.1654x; 1.1654x over previous
import functools

import jax
import jax.numpy as jnp
from jax import lax
from jax.experimental import pallas as pl
from jax.experimental.pallas import tpu as pltpu
from jax.experimental.pallas import tpu_sc as plsc

N_ATOMS = 10000
N_EDGES = 320000
D = 128
N_RBF = 20
N_INT = 3
MAX_ZN = 100
CUTOFF = 5.0
_LOG2 = 0.6931471805599453

BE = 1600
NEB = N_EDGES // BE
BA = 1000
NAB = N_ATOMS // BA

NSC = 2
NSUB = 16
EPW = N_EDGES // NSUB
CSC = 80
NCH = EPW // CSC
NWIN = 4
WINDOW = 2560
ACC_ROWS = 2688
RPT = ACC_ROWS // NSUB
OPT = WINDOW // NSUB
ZROWS = 128
EROWS = N_EDGES // D


def _ssp(x):
    return jnp.maximum(x, 0.0) + jnp.log(1.0 + jnp.exp(-jnp.abs(x))) - _LOG2


def _edge_filter_body(r_blk, fw1, fb1, fw2, fb2, w_blk):
    r = r_blk[...]
    d = jnp.sqrt(jnp.sum(r * r, axis=1, keepdims=True))
    width = CUTOFF / (N_RBF - 1)
    offs = lax.broadcasted_iota(jnp.int32, (1, N_RBF), 1).astype(jnp.float32) * width
    coeff = -0.5 / (width * width)
    fij = jnp.exp(coeff * (d - offs) ** 2)
    rcut = 0.5 * (jnp.cos(d * (jnp.pi / CUTOFF)) + 1.0)
    rcut = rcut * (d < CUTOFF).astype(jnp.float32)
    t = _ssp(jnp.dot(fij, fw1[...], preferred_element_type=jnp.float32) + fb1[...])
    w = jnp.dot(t, fw2[...], preferred_element_type=jnp.float32) + fb2[...]
    w_blk[...] = w * rcut


def _edge_filter(r_ij, fw1l, fb1l, fw2l, fb2l):
    return pl.pallas_call(
        _edge_filter_body,
        grid=(NEB,),
        in_specs=[
            pl.BlockSpec((BE, 3), lambda i: (i, 0)),
            pl.BlockSpec((N_RBF, D), lambda i: (0, 0)),
            pl.BlockSpec((1, D), lambda i: (0, 0)),
            pl.BlockSpec((D, D), lambda i: (0, 0)),
            pl.BlockSpec((1, D), lambda i: (0, 0)),
        ],
        out_specs=pl.BlockSpec((BE, D), lambda i: (i, 0)),
        out_shape=jax.ShapeDtypeStruct((N_EDGES, D), jnp.float32),
    )(r_ij, fw1l, fb1l, fw2l, fb2l)


def _prep_idx_body(ii_blk, *out_blks):
    ii = ii_blk[...]
    rows = lax.broadcasted_iota(jnp.int32, (EROWS, D), 0)
    cols = lax.broadcasted_iota(jnp.int32, (EROWS, D), 1)
    e = rows * D + cols
    dump = WINDOW + e // EPW
    for w, blk in enumerate(out_blks):
        base = w * WINDOW
        inw = (ii >= base) & (ii < base + WINDOW)
        blk[...] = jnp.where(inw, ii - base, dump)


def _prep_idx(ii2):
    return pl.pallas_call(
        _prep_idx_body,
        grid=(1,),
        in_specs=[pl.BlockSpec((EROWS, D), lambda i: (0, 0))],
        out_specs=[pl.BlockSpec((EROWS, D), lambda i: (0, 0))] * NWIN,
        out_shape=[jax.ShapeDtypeStruct((EROWS, D), jnp.int32)] * NWIN,
    )(ii2)


def _embed_body(az_blk, emb, in2f0, x0_blk, h0_blk):
    az = az_blk[...]
    ids = lax.broadcasted_iota(jnp.int32, (BA, MAX_ZN), 1)
    onehot = (az == ids).astype(jnp.float32)
    x0 = jnp.dot(onehot, emb[...], preferred_element_type=jnp.float32)
    x0_blk[...] = x0
    h0_blk[...] = jnp.dot(x0, in2f0[...], preferred_element_type=jnp.float32)


def _embed(az2, emb, in2f0):
    return pl.pallas_call(
        _embed_body,
        grid=(NAB,),
        in_specs=[
            pl.BlockSpec((BA, 1), lambda i: (i, 0)),
            pl.BlockSpec((MAX_ZN, D), lambda i: (0, 0)),
            pl.BlockSpec((D, D), lambda i: (0, 0)),
        ],
        out_specs=[
            pl.BlockSpec((BA, D), lambda i: (i, 0)),
            pl.BlockSpec((BA, D), lambda i: (i, 0)),
        ],
        out_shape=[
            jax.ShapeDtypeStruct((N_ATOMS, D), jnp.float32),
            jax.ShapeDtypeStruct((N_ATOMS, D), jnp.float32),
        ],
    )(az2, emb, in2f0)


def _node_body(agg_blk, x_blk, oW1, ob1, oW2, ob2, in2f_next, xn_blk, hn_blk):
    agg = agg_blk[...]
    t = _ssp(jnp.dot(agg, oW1[...], preferred_element_type=jnp.float32) + ob1[...])
    v = jnp.dot(t, oW2[...], preferred_element_type=jnp.float32) + ob2[...]
    xn = x_blk[...] + v
    xn_blk[...] = xn
    hn_blk[...] = jnp.dot(xn, in2f_next[...], preferred_element_type=jnp.float32)


def _node(agg, x, oW1, ob1, oW2, ob2, in2f_next):
    return pl.pallas_call(
        _node_body,
        grid=(NAB,),
        in_specs=[
            pl.BlockSpec((BA, D), lambda i: (i, 0)),
            pl.BlockSpec((BA, D), lambda i: (i, 0)),
            pl.BlockSpec((D, D), lambda i: (0, 0)),
            pl.BlockSpec((1, D), lambda i: (0, 0)),
            pl.BlockSpec((D, D), lambda i: (0, 0)),
            pl.BlockSpec((1, D), lambda i: (0, 0)),
            pl.BlockSpec((D, D), lambda i: (0, 0)),
        ],
        out_specs=[
            pl.BlockSpec((BA, D), lambda i: (i, 0)),
            pl.BlockSpec((BA, D), lambda i: (i, 0)),
        ],
        out_shape=[
            jax.ShapeDtypeStruct((N_ATOMS, D), jnp.float32),
            jax.ShapeDtypeStruct((N_ATOMS, D), jnp.float32),
        ],
    )(agg, x, oW1, ob1, oW2, ob2, in2f_next)


def _sc_body(h_hbm, w_hbm, idxj_hbm, idxc_hbm, out_hbm,
             idxj_v, idxc_v, rows_v, wij_v, zbuf, agg_sh, sem_g, sem_w):
    c = lax.axis_index("c")
    s = lax.axis_index("s")

    def zrow(i, _):
        for q in range(D // 16):
            zbuf[i, pl.ds(q * 16, 16)] = jnp.zeros((16,), jnp.float32)
        return 0
    lax.fori_loop(0, ZROWS, zrow, 0)

    pltpu.sync_copy(idxj_hbm.at[s], idxj_v)

    r0 = s * RPT
    o0 = s * OPT
    for p in range(2):
        win = 2 * c + p
        pltpu.sync_copy(idxc_hbm.at[win, s], idxc_v)
        for off, sz in ((0, 128), (128, RPT - 128)):
            pltpu.sync_copy(zbuf.at[pl.ds(0, sz)],
                            agg_sh.at[pl.ds(r0 + off, sz)])
        plsc.subcore_barrier()

        def chunk(k, _):
            cp_g = pltpu.async_copy(h_hbm.at[idxj_v.at[k]], rows_v, sem_g)
            cp_w = pltpu.async_copy(w_hbm.at[pl.ds(s * EPW + k * CSC, CSC)],
                                    wij_v, sem_w)
            cp_g.wait()
            cp_w.wait()

            def mul(r, _):
                for q in range(D // 16):
                    sl = pl.ds(q * 16, 16)
                    wij_v[r, sl] = wij_v[r, sl] * rows_v[r, sl]
                return 0
            lax.fori_loop(0, CSC, mul, 0)
            pltpu.sync_copy(wij_v, agg_sh.at[idxc_v.at[k]], add=True)
            return 0
        lax.fori_loop(0, NCH, chunk, 0)

        plsc.subcore_barrier()
        pltpu.sync_copy(agg_sh.at[pl.ds(o0, OPT)],
                        out_hbm.at[win, pl.ds(o0, OPT)])
        plsc.subcore_barrier()


def _sc_gather_scatter(h, w, idxj_r, idxc_r):
    mesh = plsc.VectorSubcoreMesh(core_axis_name="c", subcore_axis_name="s")
    k = functools.partial(
        pl.kernel,
        mesh=mesh,
        out_type=jax.ShapeDtypeStruct((NWIN, WINDOW, D), jnp.float32),
        scratch_types=[
            pltpu.VMEM((NCH, CSC), jnp.int32),
            pltpu.VMEM((NCH, CSC), jnp.int32),
            pltpu.VMEM((CSC, D), jnp.float32),
            pltpu.VMEM((CSC, D), jnp.float32),
            pltpu.VMEM((ZROWS, D), jnp.float32),
            pltpu.VMEM_SHARED((ACC_ROWS, D), jnp.float32),
            pltpu.SemaphoreType.DMA,
            pltpu.SemaphoreType.DMA,
        ],
    )(_sc_body)
    return k(h, w, idxj_r, idxc_r)


def kernel(atomic_numbers, r_ij, idx_i, idx_j, idx, idx_m, emb, in2f_W,
           fw1, fb1, fw2, fb2, o_W1, o_b1, o_W2, o_b2):
    az2 = atomic_numbers.reshape(N_ATOMS, 1).astype(jnp.int32)
    idxj_r = idx_j.astype(jnp.int32).reshape(NSUB, NCH, CSC)
    ii2 = idx_i.astype(jnp.int32).reshape(EROWS, D)
    idx_win = _prep_idx(ii2)
    idxc_r = jnp.stack([a.reshape(N_EDGES) for a in idx_win]
                       ).reshape(NWIN, NSUB, NCH, CSC)
    x, h = _embed(az2, emb, in2f_W[0])
    for l in range(N_INT):
        w = _edge_filter(r_ij, fw1[l], fb1[l].reshape(1, D),
                         fw2[l], fb2[l].reshape(1, D))
        parts = _sc_gather_scatter(h, w, idxj_r, idxc_r)
        agg = parts.reshape(NWIN * WINDOW, D)[:N_ATOMS]
        x, h = _node(agg, x, o_W1[l], o_b1[l].reshape(1, D),
                     o_W2[l], o_b2[l].reshape(1, D),
                     in2f_W[(l + 1) % N_INT])
    return x
